# baseline (speedup 1.0000x reference)
import functools

import jax
import jax.numpy as jnp
from jax import lax
from jax.experimental import pallas as pl
from jax.experimental.pallas import tpu as pltpu
from jax.experimental.pallas import tpu_sc as plsc

B, C, H, W = 64, 768, 32, 32
HW = H * W
HALF = C // 2

NC, NS, L = 2, 16, 16
NW = NC * NS
CPW = C // NW
ROW_W = HALF // CPW
FLAT = CPW * HW
CHUNKS = FLAT // L


def _in_copy(fm_hbm, buf, sem, b, c0):
    off = (b * C + c0) * HW
    return pltpu.make_async_copy(fm_hbm.at[pl.ds(off, FLAT)], buf, sem)


def _out_copy(out_hbm, buf, sem, b, c0):
    off = (b * C + c0) * HW
    return pltpu.make_async_copy(buf, out_hbm.at[pl.ds(off, FLAT)], sem)


def _sc_body(emb_hbm, fm_hbm, out_hbm, pos, in0, in1, out0, out1, embv,
             s_in0, s_in1, s_out0, s_out1):
    wid = lax.axis_index("s") * NC + lax.axis_index("c")
    c0 = wid * CPW
    isr = (wid < ROW_W).astype(jnp.int32)

    pltpu.sync_copy(emb_hbm.at[pl.ds(c0 * H, CPW * H)], embv)

    lane = lax.iota(jnp.int32, L)

    def _pos_step(t, _):
        j = t >> 6
        q = t & 63
        idx_row = jnp.full((L,), j * H + (q >> 1), dtype=jnp.int32)
        idx_col = j * H + ((q & 1) << 4) + lane
        idx = isr * idx_row + (1 - isr) * idx_col
        pos[pl.ds(t * L, L)] = plsc.load_gather(embv, [idx])
        return ()

    lax.fori_loop(0, CHUNKS, _pos_step, (), unroll=2)

    def _add(dst, src):
        def step(q, _):
            o = q * L
            dst[pl.ds(o, L)] = src[pl.ds(o, L)] + pos[pl.ds(o, L)]
            return ()
        lax.fori_loop(0, CHUNKS, step, (), unroll=8)

    _in_copy(fm_hbm, in0, s_in0, 0, c0).start()
    _in_copy(fm_hbm, in1, s_in1, 1, c0).start()

    def _pair(t, _):
        b0 = 2 * t
        b1 = b0 + 1

        _in_copy(fm_hbm, in0, s_in0, b0, c0).wait()

        @pl.when(t > 0)
        def _():
            _out_copy(out_hbm, out0, s_out0, b0 - 2, c0).wait()

        _add(out0, in0)
        _out_copy(out_hbm, out0, s_out0, b0, c0).start()

        @pl.when(b0 + 2 < B)
        def _():
            _in_copy(fm_hbm, in0, s_in0, b0 + 2, c0).start()

        _in_copy(fm_hbm, in1, s_in1, b1, c0).wait()

        @pl.when(t > 0)
        def _():
            _out_copy(out_hbm, out1, s_out1, b1 - 2, c0).wait()

        _add(out1, in1)
        _out_copy(out_hbm, out1, s_out1, b1, c0).start()

        @pl.when(b1 + 2 < B)
        def _():
            _in_copy(fm_hbm, in1, s_in1, b1 + 2, c0).start()

        return ()

    lax.fori_loop(0, B // 2, _pair, ())

    _out_copy(out_hbm, out0, s_out0, B - 2, c0).wait()
    _out_copy(out_hbm, out1, s_out1, B - 1, c0).wait()


def kernel(feature_map, row_embed, col_embed):
    emb = jnp.concatenate([row_embed.T, col_embed.T], axis=0).reshape(-1)
    fm_flat = feature_map.reshape(-1)

    mesh = plsc.VectorSubcoreMesh(
        core_axis_name="c", subcore_axis_name="s", num_cores=NC, num_subcores=NS
    )
    run = pl.kernel(
        _sc_body,
        out_type=jax.ShapeDtypeStruct((B * C * HW,), jnp.float32),
        mesh=mesh,
        compiler_params=pltpu.CompilerParams(needs_layout_passes=False),
        scratch_types=[
            pltpu.VMEM((FLAT,), jnp.float32),
            pltpu.VMEM((FLAT,), jnp.float32),
            pltpu.VMEM((FLAT,), jnp.float32),
            pltpu.VMEM((FLAT,), jnp.float32),
            pltpu.VMEM((FLAT,), jnp.float32),
            pltpu.VMEM((CPW * H,), jnp.float32),
            pltpu.SemaphoreType.DMA,
            pltpu.SemaphoreType.DMA,
            pltpu.SemaphoreType.DMA,
            pltpu.SemaphoreType.DMA,
        ],
    )
    out = run(emb, fm_flat)
    return out.reshape(B, C, H, W)


# speedup vs baseline: 1.8830x; 1.8830x over previous
import jax
import jax.numpy as jnp
from jax import lax
from jax.experimental import pallas as pl
from jax.experimental.pallas import tpu as pltpu
from jax.experimental.pallas import tpu_sc as plsc

B, C, H, W = 64, 768, 32, 32
HW = H * W
HALF = C // 2

NC, NS, L = 2, 16, 16
NW = NC * NS
CPW = C // NW
ROW_W = HALF // CPW
CHUNKS = CPW * HW // L


def _in_copy(fm_hbm, buf, sem, b, c0):
    return pltpu.make_async_copy(
        fm_hbm.at[pl.ds(b * C + c0, CPW)], buf, sem
    )


def _out_copy(out_hbm, buf, sem, b, c0):
    return pltpu.make_async_copy(
        buf, out_hbm.at[pl.ds(b * C + c0, CPW)], sem
    )


def _sc_body(emb_hbm, fm_hbm, out_hbm, pos, in0, in1, out0, out1, embv,
             s_in0, s_in1, s_out0, s_out1):
    wid = lax.axis_index("s") * NC + lax.axis_index("c")
    c0 = wid * CPW
    isr = (wid < ROW_W).astype(jnp.int32)

    pltpu.sync_copy(emb_hbm.at[pl.ds(c0 * H, CPW * H)], embv)

    lane = lax.iota(jnp.int32, L)

    @plsc.parallel_loop(0, CHUNKS, unroll=4)
    def _pos_step(t):
        j = t >> 6
        q = t & 63
        idx_row = jnp.full((L,), j * H + (q >> 1), dtype=jnp.int32)
        idx_col = j * H + ((q & 1) << 4) + lane
        idx = isr * idx_row + (1 - isr) * idx_col
        pos[j, pl.ds((q & 63) * L, L)] = plsc.load_gather(embv, [idx])

    def _add(dst, src):
        @plsc.parallel_loop(0, CHUNKS, unroll=8)
        def _step(t):
            j = t >> 6
            o = (t & 63) * L
            dst[j, pl.ds(o, L)] = src[j, pl.ds(o, L)] + pos[j, pl.ds(o, L)]

    _in_copy(fm_hbm, in0, s_in0, 0, c0).start()
    _in_copy(fm_hbm, in1, s_in1, 1, c0).start()

    def _pair(t, _):
        b0 = 2 * t
        b1 = b0 + 1

        _in_copy(fm_hbm, in0, s_in0, b0, c0).wait()

        @pl.when(t > 0)
        def _():
            _out_copy(out_hbm, out0, s_out0, b0 - 2, c0).wait()

        _add(out0, in0)
        _out_copy(out_hbm, out0, s_out0, b0, c0).start()

        @pl.when(b0 + 2 < B)
        def _():
            _in_copy(fm_hbm, in0, s_in0, b0 + 2, c0).start()

        _in_copy(fm_hbm, in1, s_in1, b1, c0).wait()

        @pl.when(t > 0)
        def _():
            _out_copy(out_hbm, out1, s_out1, b1 - 2, c0).wait()

        _add(out1, in1)
        _out_copy(out_hbm, out1, s_out1, b1, c0).start()

        @pl.when(b1 + 2 < B)
        def _():
            _in_copy(fm_hbm, in1, s_in1, b1 + 2, c0).start()

        return ()

    lax.fori_loop(0, B // 2, _pair, ())

    _out_copy(out_hbm, out0, s_out0, B - 2, c0).wait()
    _out_copy(out_hbm, out1, s_out1, B - 1, c0).wait()


def kernel(feature_map, row_embed, col_embed):
    emb = jnp.concatenate([row_embed.T, col_embed.T], axis=0).reshape(-1)
    fm2 = feature_map.reshape(B * C, HW)

    mesh = plsc.VectorSubcoreMesh(
        core_axis_name="c", subcore_axis_name="s", num_cores=NC, num_subcores=NS
    )
    run = pl.kernel(
        _sc_body,
        out_type=jax.ShapeDtypeStruct((B * C, HW), jnp.float32),
        mesh=mesh,
        compiler_params=pltpu.CompilerParams(
            needs_layout_passes=False, use_tc_tiling_on_sc=True
        ),
        scratch_types=[
            pltpu.VMEM((CPW, HW), jnp.float32),
            pltpu.VMEM((CPW, HW), jnp.float32),
            pltpu.VMEM((CPW, HW), jnp.float32),
            pltpu.VMEM((CPW, HW), jnp.float32),
            pltpu.VMEM((CPW, HW), jnp.float32),
            pltpu.VMEM((CPW * H,), jnp.float32),
            pltpu.SemaphoreType.DMA,
            pltpu.SemaphoreType.DMA,
            pltpu.SemaphoreType.DMA,
            pltpu.SemaphoreType.DMA,
        ],
    )
    out = run(emb, fm2)
    return out.reshape(B, C, H, W)


# speedup vs baseline: 4.2495x; 2.2568x over previous
import jax
import jax.numpy as jnp
from jax import lax
from jax.experimental import pallas as pl
from jax.experimental.pallas import tpu as pltpu

B, C, H, W = 64, 768, 32, 32
HW = H * W
HALF = C // 2

B_BLK = 2


def _body(emb_ref, fm_ref, out_ref, pos_ref):
    i = pl.program_id(0)

    @pl.when(i == 0)
    def _build_pos():
        e = emb_ref[...]
        ii = lax.broadcasted_iota(jnp.int32, (H, HW), 0)
        jj = lax.broadcasted_iota(jnp.int32, (H, HW), 1)
        sel_row = ((jj // W) == ii).astype(jnp.float32)
        sel_col = ((jj % W) == ii).astype(jnp.float32)
        pos_ref[:HALF] = lax.dot(
            e[:HALF], sel_row,
            precision=lax.Precision.HIGHEST,
            preferred_element_type=jnp.float32,
        )
        pos_ref[HALF:] = lax.dot(
            e[HALF:], sel_col,
            precision=lax.Precision.HIGHEST,
            preferred_element_type=jnp.float32,
        )

    out_ref[...] = fm_ref[...] + pos_ref[...][None]


def kernel(feature_map, row_embed, col_embed):
    emb = jnp.concatenate([row_embed.T, col_embed.T], axis=0)
    fm3 = feature_map.reshape(B, C, HW)

    out = pl.pallas_call(
        _body,
        grid=(B // B_BLK,),
        in_specs=[
            pl.BlockSpec((C, H), lambda i: (0, 0)),
            pl.BlockSpec((B_BLK, C, HW), lambda i: (i, 0, 0)),
        ],
        out_specs=pl.BlockSpec((B_BLK, C, HW), lambda i: (i, 0, 0)),
        out_shape=jax.ShapeDtypeStruct((B, C, HW), jnp.float32),
        scratch_shapes=[pltpu.VMEM((C, HW), jnp.float32)],
    )(emb, fm3)
    return out.reshape(B, C, H, W)


# speedup vs baseline: 4.2640x; 1.0034x over previous
import jax
import jax.numpy as jnp
from jax import lax
from jax.experimental import pallas as pl
from jax.experimental.pallas import tpu as pltpu

B, C, H, W = 64, 768, 32, 32
HW = H * W
HALF = C // 2

B_BLK = 4


def _body(emb_ref, fm_ref, out_ref, pos_ref):
    i = pl.program_id(0)

    @pl.when(i == 0)
    def _build_pos():
        e = emb_ref[...]
        ii = lax.broadcasted_iota(jnp.int32, (H, HW), 0)
        jj = lax.broadcasted_iota(jnp.int32, (H, HW), 1)
        sel_row = ((jj // W) == ii).astype(jnp.float32)
        sel_col = ((jj % W) == ii).astype(jnp.float32)
        pos_ref[:HALF] = lax.dot(
            e[:HALF], sel_row,
            precision=lax.Precision.HIGHEST,
            preferred_element_type=jnp.float32,
        )
        pos_ref[HALF:] = lax.dot(
            e[HALF:], sel_col,
            precision=lax.Precision.HIGHEST,
            preferred_element_type=jnp.float32,
        )

    out_ref[...] = fm_ref[...] + pos_ref[...][None]


def kernel(feature_map, row_embed, col_embed):
    emb = jnp.concatenate([row_embed.T, col_embed.T], axis=0)
    fm3 = feature_map.reshape(B, C, HW)

    out = pl.pallas_call(
        _body,
        grid=(B // B_BLK,),
        in_specs=[
            pl.BlockSpec((C, H), lambda i: (0, 0)),
            pl.BlockSpec((B_BLK, C, HW), lambda i: (i, 0, 0)),
        ],
        out_specs=pl.BlockSpec((B_BLK, C, HW), lambda i: (i, 0, 0)),
        out_shape=jax.ShapeDtypeStruct((B, C, HW), jnp.float32),
        scratch_shapes=[pltpu.VMEM((C, HW), jnp.float32)],
    )(emb, fm3)
    return out.reshape(B, C, H, W)
